# Initial kernel scaffold; baseline (speedup 1.0000x reference)
import functools

import jax
import jax.numpy as jnp
from jax import lax
from jax.experimental import pallas as pl
from jax.experimental.pallas import tpu as pltpu
from jax.experimental.pallas import tpu_sc as plsc

NV = 25000
NC = 25000
DEG = 16
OUT = 64
INIT_IN = 128
NW = 32
RB = 784
NVP = NW * RB
NT = 2 * NVP
CB = 112
LANES = 16


def _init_body(x_ref, w_ref, b_ref, last_ref, base_ref):
    h = jnp.dot(x_ref[...], w_ref[0], preferred_element_type=jnp.float32)
    h = h + b_ref[0]
    last_ref[...] = h[:, :OUT]
    base_ref[...] = h[:, OUT:]


_init_call = pl.pallas_call(
    _init_body,
    grid=(NT // RB,),
    in_specs=[
        pl.BlockSpec((RB, INIT_IN), lambda i: (i, 0)),
        pl.BlockSpec((1, INIT_IN, 2 * OUT), lambda i: (i // (NVP // RB), 0, 0)),
        pl.BlockSpec((1, 1, 2 * OUT), lambda i: (i // (NVP // RB), 0, 0)),
    ],
    out_specs=[
        pl.BlockSpec((RB, OUT), lambda i: (i, 0)),
        pl.BlockSpec((RB, OUT), lambda i: (i, 0)),
    ],
    out_shape=[
        jax.ShapeDtypeStruct((NT, OUT), jnp.float32),
        jax.ShapeDtypeStruct((NT, OUT), jnp.float32),
    ],
)


def _round_body(a_ref, base_ref, w_ref, t_ref, d_ref):
    h = jnp.dot(a_ref[...], w_ref[0], preferred_element_type=jnp.float32)
    t_ref[...] = h[:, :OUT]
    d_ref[...] = h[:, OUT:] + base_ref[...]


_round_call = pl.pallas_call(
    _round_body,
    grid=(NT // RB,),
    in_specs=[
        pl.BlockSpec((RB, OUT), lambda i: (i, 0)),
        pl.BlockSpec((RB, OUT), lambda i: (i, 0)),
        pl.BlockSpec((1, OUT, 2 * OUT), lambda i: (i // (NVP // RB), 0, 0)),
    ],
    out_specs=[
        pl.BlockSpec((RB, OUT), lambda i: (i, 0)),
        pl.BlockSpec((RB, OUT), lambda i: (i, 0)),
    ],
    out_shape=[
        jax.ShapeDtypeStruct((NT, OUT), jnp.float32),
        jax.ShapeDtypeStruct((NT, OUT), jnp.float32),
    ],
)


def _colsum_body(v_ref, o_ref):
    i = pl.program_id(0)

    @pl.when(i == 0)
    def _():
        o_ref[...] = jnp.zeros_like(o_ref)

    rows = lax.broadcasted_iota(jnp.int32, (RB, OUT), 0) + i * RB
    x = jnp.where(rows < NV, v_ref[...], 0.0)
    o_ref[0:1, 0:OUT] = o_ref[0:1, 0:OUT] + jnp.sum(x, axis=0, keepdims=True)


_colsum_call = pl.pallas_call(
    _colsum_body,
    grid=(NVP // RB,),
    in_specs=[pl.BlockSpec((RB, OUT), lambda i: (i, 0))],
    out_specs=pl.BlockSpec((8, 128), lambda i: (0, 0)),
    out_shape=jax.ShapeDtypeStruct((8, 128), jnp.float32),
)


def _q_body(agg_ref, wq_ref, bq_ref, v_ref, q_ref):
    s = jnp.dot(agg_ref[0:1, 0:OUT], wq_ref[0:OUT, :],
                preferred_element_type=jnp.float32)
    q_ref[...] = jnp.dot(v_ref[...], wq_ref[OUT:, :],
                         preferred_element_type=jnp.float32) + (s[0, 0] + bq_ref[0, 0])


_q_call = pl.pallas_call(
    _q_body,
    grid=(NVP // RB,),
    in_specs=[
        pl.BlockSpec((8, 128), lambda i: (0, 0)),
        pl.BlockSpec((2 * OUT, 1), lambda i: (0, 0)),
        pl.BlockSpec((1, 1), lambda i: (0, 0)),
        pl.BlockSpec((RB, OUT), lambda i: (i, 0)),
    ],
    out_specs=pl.BlockSpec((RB, 1), lambda i: (i, 0)),
    out_shape=jax.ShapeDtypeStruct((NVP, 1), jnp.float32),
)


def _make_gathersum(n_out):
    bw = n_out // NW
    nchunks = bw // CB
    mesh = plsc.VectorSubcoreMesh(core_axis_name="c", subcore_axis_name="s")

    @functools.partial(
        pl.kernel,
        out_type=jax.ShapeDtypeStruct((n_out, OUT), jnp.float32),
        mesh=mesh,
        scratch_types=[
            pltpu.VMEM((DEG, CB), jnp.int32),
            pltpu.VMEM((CB, OUT), jnp.float32),
            pltpu.VMEM((DEG, CB, OUT), jnp.float32),
            pltpu.SemaphoreType.DMA,
        ],
    )
    def gathersum(table, idxt, dense, out, idx_v, acc_v, gat_v, sem):
        wid = lax.axis_index("s") * 2 + lax.axis_index("c")
        base = wid * bw

        def chunk_body(t, carry):
            off = base + t * CB
            pltpu.sync_copy(idxt.at[:, pl.ds(off, CB)], idx_v)
            pltpu.sync_copy(dense.at[pl.ds(off, CB)], acc_v)
            copies = [
                pltpu.async_copy(table.at[idx_v.at[j]], gat_v.at[j], sem)
                for j in range(DEG)
            ]
            for c in copies:
                c.wait()

            def row_body(r, rc):
                for c in range(OUT // LANES):
                    sl = pl.ds(c * LANES, LANES)
                    v = acc_v[r, sl]
                    for j in range(DEG):
                        v = v + gat_v[j, r, sl]
                    acc_v[r, sl] = v
                return rc

            lax.fori_loop(0, CB, row_body, 0)
            pltpu.sync_copy(acc_v, out.at[pl.ds(off, CB)])
            return carry

        lax.fori_loop(0, nchunks, chunk_body, 0)

    return gathersum


_gathersum_full = _make_gathersum(NT)
_gathersum_half = _make_gathersum(NVP)


def kernel(x, var_constr_index, constr_var_index, W_iv, b_iv, W_ic, b_ic,
           W_v, b_v, W_c, b_c, W_q, b_q):
    pad = NVP - NV
    xv = jnp.pad(x[:NV], ((0, pad), (0, 0)))
    xc = jnp.pad(x[NV:], ((0, pad), (0, 0)))
    xp = jnp.concatenate([xv, xc], axis=0)

    Wc1, Wc2, Wc3 = W_c[:OUT], W_c[OUT:2 * OUT], W_c[2 * OUT:]
    Wv1, Wv2, Wv3 = W_v[:OUT], W_v[OUT:2 * OUT], W_v[2 * OUT:]

    W0 = jnp.stack([
        jnp.concatenate([W_iv, Wv3], axis=1),
        jnp.concatenate([W_ic, Wc3], axis=1),
    ])
    B0 = jnp.stack([
        jnp.concatenate([b_iv, b_v])[None, :],
        jnp.concatenate([b_ic, b_c])[None, :],
    ])
    W1 = jnp.stack([
        jnp.concatenate([Wc1, Wv2], axis=1),
        jnp.concatenate([Wv1, Wc2], axis=1),
    ])
    w2 = jnp.concatenate([Wv1, Wv2], axis=1)
    W2 = jnp.stack([w2, w2])

    idxt = jnp.concatenate([
        jnp.pad(var_constr_index + NVP, ((0, pad), (0, 0))),
        jnp.pad(constr_var_index, ((0, pad), (0, 0))),
    ], axis=0).T.astype(jnp.int32)

    last0, base0 = _init_call(xp, W0, B0)
    t1, d1 = _round_call(last0, base0, W1)
    l1 = _gathersum_full(t1, idxt, d1)
    t2, d2 = _round_call(l1, base0, W2)
    l2 = _gathersum_half(t2, idxt, d2)
    agg = _colsum_call(l2)
    q = _q_call(agg, W_q, jnp.reshape(b_q, (1, 1)), l2)
    return q[:NV]


# speedup vs baseline: 3.5426x; 3.5426x over previous
import functools

import jax
import jax.numpy as jnp
from jax import lax
from jax.experimental import pallas as pl
from jax.experimental.pallas import tpu as pltpu
from jax.experimental.pallas import tpu_sc as plsc

NV = 25000
NC = 25000
DEG = 16
OUT = 64
INIT_IN = 128
NW = 32
RB = 784
NVP = NW * RB
NT = 2 * NVP
CB = 112
LANES = 16


def _init_body(x_ref, w_ref, b_ref, last_ref, base_ref):
    h = jnp.dot(x_ref[...], w_ref[0], preferred_element_type=jnp.float32, precision=lax.Precision.HIGHEST)
    h = h + b_ref[0]
    last_ref[...] = h[:, :OUT]
    base_ref[...] = h[:, OUT:]


_init_call = pl.pallas_call(
    _init_body,
    grid=(NT // RB,),
    in_specs=[
        pl.BlockSpec((RB, INIT_IN), lambda i: (i, 0)),
        pl.BlockSpec((1, INIT_IN, 2 * OUT), lambda i: (i // (NVP // RB), 0, 0)),
        pl.BlockSpec((1, 1, 2 * OUT), lambda i: (i // (NVP // RB), 0, 0)),
    ],
    out_specs=[
        pl.BlockSpec((RB, OUT), lambda i: (i, 0)),
        pl.BlockSpec((RB, OUT), lambda i: (i, 0)),
    ],
    out_shape=[
        jax.ShapeDtypeStruct((NT, OUT), jnp.float32),
        jax.ShapeDtypeStruct((NT, OUT), jnp.float32),
    ],
)


def _round_body(a_ref, base_ref, w_ref, t_ref, d_ref):
    h = jnp.dot(a_ref[...], w_ref[0], preferred_element_type=jnp.float32, precision=lax.Precision.HIGHEST)
    t_ref[...] = h[:, :OUT]
    d_ref[...] = h[:, OUT:] + base_ref[...]


_round_call = pl.pallas_call(
    _round_body,
    grid=(NT // RB,),
    in_specs=[
        pl.BlockSpec((RB, OUT), lambda i: (i, 0)),
        pl.BlockSpec((RB, OUT), lambda i: (i, 0)),
        pl.BlockSpec((1, OUT, 2 * OUT), lambda i: (i // (NVP // RB), 0, 0)),
    ],
    out_specs=[
        pl.BlockSpec((RB, OUT), lambda i: (i, 0)),
        pl.BlockSpec((RB, OUT), lambda i: (i, 0)),
    ],
    out_shape=[
        jax.ShapeDtypeStruct((NT, OUT), jnp.float32),
        jax.ShapeDtypeStruct((NT, OUT), jnp.float32),
    ],
)


def _colsum_body(v_ref, o_ref):
    i = pl.program_id(0)

    @pl.when(i == 0)
    def _():
        o_ref[...] = jnp.zeros_like(o_ref)

    rows = lax.broadcasted_iota(jnp.int32, (RB, OUT), 0) + i * RB
    x = jnp.where(rows < NV, v_ref[...], 0.0)
    o_ref[0:1, 0:OUT] = o_ref[0:1, 0:OUT] + jnp.sum(x, axis=0, keepdims=True)


_colsum_call = pl.pallas_call(
    _colsum_body,
    grid=(NVP // RB,),
    in_specs=[pl.BlockSpec((RB, OUT), lambda i: (i, 0))],
    out_specs=pl.BlockSpec((8, 128), lambda i: (0, 0)),
    out_shape=jax.ShapeDtypeStruct((8, 128), jnp.float32),
)


def _q_body(agg_ref, wq_ref, bq_ref, v_ref, q_ref):
    s = jnp.dot(agg_ref[0:1, 0:OUT], wq_ref[0:OUT, :],
                preferred_element_type=jnp.float32, precision=lax.Precision.HIGHEST)
    q_ref[...] = jnp.dot(v_ref[...], wq_ref[OUT:, :],
                         preferred_element_type=jnp.float32, precision=lax.Precision.HIGHEST) + (s[0, 0] + bq_ref[0, 0])


_q_call = pl.pallas_call(
    _q_body,
    grid=(NVP // RB,),
    in_specs=[
        pl.BlockSpec((8, 128), lambda i: (0, 0)),
        pl.BlockSpec((2 * OUT, 1), lambda i: (0, 0)),
        pl.BlockSpec((1, 1), lambda i: (0, 0)),
        pl.BlockSpec((RB, OUT), lambda i: (i, 0)),
    ],
    out_specs=pl.BlockSpec((RB, 1), lambda i: (i, 0)),
    out_shape=jax.ShapeDtypeStruct((NVP, 1), jnp.float32),
)


def _make_gathersum(n_out):
    bw = n_out // NW
    nchunks = bw // CB
    mesh = plsc.VectorSubcoreMesh(core_axis_name="c", subcore_axis_name="s")

    @functools.partial(
        pl.kernel,
        out_type=jax.ShapeDtypeStruct((n_out, OUT), jnp.float32),
        mesh=mesh,
        scratch_types=[
            pltpu.VMEM((DEG, CB), jnp.int32),
            pltpu.VMEM((CB, OUT), jnp.float32),
            pltpu.VMEM((DEG, CB, OUT), jnp.float32),
            pltpu.SemaphoreType.DMA,
        ],
        compiler_params=pltpu.CompilerParams(use_tc_tiling_on_sc=False),
    )
    def gathersum(table, idxt, dense, out, idx_v, acc_v, gat_v, sem):
        wid = lax.axis_index("s") * 2 + lax.axis_index("c")
        base = wid * bw

        def chunk_body(t, carry):
            off = base + t * CB
            pltpu.sync_copy(idxt.at[:, pl.ds(off, CB)], idx_v)
            pltpu.sync_copy(dense.at[pl.ds(off, CB)], acc_v)
            copies = [
                pltpu.async_copy(table.at[idx_v.at[j]], gat_v.at[j], sem)
                for j in range(DEG)
            ]
            for c in copies:
                c.wait()

            def row_body(r, rc):
                for c in range(OUT // LANES):
                    sl = pl.ds(c * LANES, LANES)
                    v = acc_v[r, sl]
                    for j in range(DEG):
                        v = v + gat_v[j, r, sl]
                    acc_v[r, sl] = v
                return rc

            lax.fori_loop(0, CB, row_body, 0)
            pltpu.sync_copy(acc_v, out.at[pl.ds(off, CB)])
            return carry

        lax.fori_loop(0, nchunks, chunk_body, 0)

    return gathersum


_gathersum_full = _make_gathersum(NT)
_gathersum_half = _make_gathersum(NVP)


def kernel(x, var_constr_index, constr_var_index, W_iv, b_iv, W_ic, b_ic,
           W_v, b_v, W_c, b_c, W_q, b_q):
    pad = NVP - NV
    xv = jnp.pad(x[:NV], ((0, pad), (0, 0)))
    xc = jnp.pad(x[NV:], ((0, pad), (0, 0)))
    xp = jnp.concatenate([xv, xc], axis=0)

    Wc1, Wc2, Wc3 = W_c[:OUT], W_c[OUT:2 * OUT], W_c[2 * OUT:]
    Wv1, Wv2, Wv3 = W_v[:OUT], W_v[OUT:2 * OUT], W_v[2 * OUT:]

    W0 = jnp.stack([
        jnp.concatenate([W_iv, Wv3], axis=1),
        jnp.concatenate([W_ic, Wc3], axis=1),
    ])
    B0 = jnp.stack([
        jnp.concatenate([b_iv, b_v])[None, :],
        jnp.concatenate([b_ic, b_c])[None, :],
    ])
    W1 = jnp.stack([
        jnp.concatenate([Wc1, Wv2], axis=1),
        jnp.concatenate([Wv1, Wc2], axis=1),
    ])
    w2 = jnp.concatenate([Wv1, Wv2], axis=1)
    W2 = jnp.stack([w2, w2])

    idxt = jnp.concatenate([
        jnp.pad(var_constr_index + NVP, ((0, pad), (0, 0))),
        jnp.pad(constr_var_index, ((0, pad), (0, 0))),
    ], axis=0).T.astype(jnp.int32)

    last0, base0 = _init_call(xp, W0, B0)
    t1, d1 = _round_call(last0, base0, W1)
    l1 = _gathersum_full(t1, idxt, d1)
    t2, d2 = _round_call(l1, base0, W2)
    l2 = _gathersum_half(t2, idxt, d2)
    agg = _colsum_call(l2)
    q = _q_call(agg, W_q, jnp.reshape(b_q, (1, 1)), l2)
    return q[:NV]


# speedup vs baseline: 4.1369x; 1.1678x over previous
import functools

import jax
import jax.numpy as jnp
from jax import lax
from jax.experimental import pallas as pl
from jax.experimental.pallas import tpu as pltpu
from jax.experimental.pallas import tpu_sc as plsc

NV = 25000
NC = 25000
DEG = 16
OUT = 64
INIT_IN = 128
NW = 32
RB = 784
NVP = NW * RB
NT = 2 * NVP
CB = 112
LANES = 16


def _init_body(x_ref, w_ref, b_ref, last_ref):
    last_ref[...] = jnp.dot(x_ref[...], w_ref[0],
                            preferred_element_type=jnp.float32) + b_ref[0]


_init_call = pl.pallas_call(
    _init_body,
    grid=(NT // RB,),
    in_specs=[
        pl.BlockSpec((RB, INIT_IN), lambda i: (i, 0)),
        pl.BlockSpec((1, INIT_IN, OUT), lambda i: (i // (NVP // RB), 0, 0)),
        pl.BlockSpec((1, 1, OUT), lambda i: (i // (NVP // RB), 0, 0)),
    ],
    out_specs=pl.BlockSpec((RB, OUT), lambda i: (i, 0)),
    out_shape=jax.ShapeDtypeStruct((NT, OUT), jnp.float32),
)


def _update_body(agg_ref, last_ref, raw_ref, w_ref, b_ref, o_ref):
    cat = jnp.concatenate([agg_ref[...], last_ref[...], raw_ref[...]], axis=1)
    o_ref[...] = jnp.dot(cat, w_ref[0], preferred_element_type=jnp.float32) + b_ref[0]


def _make_update(n_rows):
    return pl.pallas_call(
        _update_body,
        grid=(n_rows // RB,),
        in_specs=[
            pl.BlockSpec((RB, OUT), lambda i: (i, 0)),
            pl.BlockSpec((RB, OUT), lambda i: (i, 0)),
            pl.BlockSpec((RB, INIT_IN), lambda i: (i, 0)),
            pl.BlockSpec((1, 2 * OUT + INIT_IN, OUT),
                         lambda i: (i // (NVP // RB), 0, 0)),
            pl.BlockSpec((1, 1, OUT), lambda i: (i // (NVP // RB), 0, 0)),
        ],
        out_specs=pl.BlockSpec((RB, OUT), lambda i: (i, 0)),
        out_shape=jax.ShapeDtypeStruct((n_rows, OUT), jnp.float32),
    )


_update_full = _make_update(NT)
_update_half = _make_update(NVP)


def _colsum_body(v_ref, o_ref):
    i = pl.program_id(0)

    @pl.when(i == 0)
    def _():
        o_ref[...] = jnp.zeros_like(o_ref)

    rows = lax.broadcasted_iota(jnp.int32, (RB, OUT), 0) + i * RB
    x = jnp.where(rows < NV, v_ref[...], 0.0)
    o_ref[0:1, 0:OUT] = o_ref[0:1, 0:OUT] + jnp.sum(x, axis=0, keepdims=True)


_colsum_call = pl.pallas_call(
    _colsum_body,
    grid=(NVP // RB,),
    in_specs=[pl.BlockSpec((RB, OUT), lambda i: (i, 0))],
    out_specs=pl.BlockSpec((8, 128), lambda i: (0, 0)),
    out_shape=jax.ShapeDtypeStruct((8, 128), jnp.float32),
)


def _q_body(agg_ref, wq_ref, bq_ref, v_ref, q_ref):
    agg_rep = jnp.broadcast_to(agg_ref[0:1, 0:OUT], (RB, OUT))
    cat = jnp.concatenate([agg_rep, v_ref[...]], axis=1)
    q_ref[...] = jnp.dot(cat, wq_ref[...],
                         preferred_element_type=jnp.float32) + bq_ref[0, 0]


_q_call = pl.pallas_call(
    _q_body,
    grid=(NVP // RB,),
    in_specs=[
        pl.BlockSpec((8, 128), lambda i: (0, 0)),
        pl.BlockSpec((2 * OUT, 1), lambda i: (0, 0)),
        pl.BlockSpec((1, 1), lambda i: (0, 0)),
        pl.BlockSpec((RB, OUT), lambda i: (i, 0)),
    ],
    out_specs=pl.BlockSpec((RB, 1), lambda i: (i, 0)),
    out_shape=jax.ShapeDtypeStruct((NVP, 1), jnp.float32),
)


def _make_gathersum(n_out):
    bw = n_out // NW
    nchunks = bw // CB
    half = DEG // 2
    mesh = plsc.VectorSubcoreMesh(core_axis_name="c", subcore_axis_name="s")

    @functools.partial(
        pl.kernel,
        out_type=jax.ShapeDtypeStruct((n_out, OUT), jnp.float32),
        mesh=mesh,
        scratch_types=[
            pltpu.VMEM((DEG, CB), jnp.int32),
            pltpu.VMEM((CB, OUT), jnp.float32),
            pltpu.VMEM((DEG, CB, OUT), jnp.float32),
            pltpu.SemaphoreType.DMA,
            pltpu.SemaphoreType.DMA,
            pltpu.SemaphoreType.DMA,
        ],
        compiler_params=pltpu.CompilerParams(use_tc_tiling_on_sc=False),
    )
    def gathersum(table, idxt, out, idx_v, acc_v, gat_v,
                  isem, gsem_a, gsem_b):
        wid = lax.axis_index("s") * 2 + lax.axis_index("c")
        base = wid * bw

        def accum(j0, j1, init):
            def row_body(r, rc):
                for c in range(OUT // LANES):
                    sl = pl.ds(c * LANES, LANES)
                    v = gat_v[j0, r, sl] if init else acc_v[r, sl]
                    for j in range(j0 + (1 if init else 0), j1):
                        v = v + gat_v[j, r, sl]
                    acc_v[r, sl] = v
                return rc

            lax.fori_loop(0, CB, row_body, 0, unroll=2)

        def chunk_body(t, carry):
            off = base + t * CB
            ci = pltpu.async_copy(idxt.at[:, pl.ds(off, CB)], idx_v, isem)
            ci.wait()
            ha = [pltpu.async_copy(table.at[idx_v.at[j]], gat_v.at[j], gsem_a)
                  for j in range(half)]
            hb = [pltpu.async_copy(table.at[idx_v.at[j]], gat_v.at[j], gsem_b)
                  for j in range(half, DEG)]
            for h in ha:
                h.wait()
            accum(0, half, True)
            for h in hb:
                h.wait()
            accum(half, DEG, False)
            pltpu.sync_copy(acc_v, out.at[pl.ds(off, CB)])
            return carry

        lax.fori_loop(0, nchunks, chunk_body, 0)

    return gathersum


_gathersum_full = _make_gathersum(NT)
_gathersum_half = _make_gathersum(NVP)


def kernel(x, var_constr_index, constr_var_index, W_iv, b_iv, W_ic, b_ic,
           W_v, b_v, W_c, b_c, W_q, b_q):
    pad = NVP - NV
    xv = jnp.pad(x[:NV], ((0, pad), (0, 0)))
    xc = jnp.pad(x[NV:], ((0, pad), (0, 0)))
    xp = jnp.concatenate([xv, xc], axis=0)

    Wi = jnp.stack([W_iv, W_ic])
    Bi = jnp.stack([b_iv[None, :], b_ic[None, :]])
    Wu = jnp.stack([W_v, W_c])
    Bu = jnp.stack([b_v[None, :], b_c[None, :]])

    idxt = jnp.concatenate([
        jnp.pad(var_constr_index + NVP, ((0, pad), (0, 0))),
        jnp.pad(constr_var_index, ((0, pad), (0, 0))),
    ], axis=0).T.astype(jnp.int32)

    last0 = _init_call(xp, Wi, Bi)
    agg1 = _gathersum_full(last0, idxt)
    l1 = _update_full(agg1, last0, xp, Wu, Bu)
    agg2 = _gathersum_half(l1, idxt)
    l2 = _update_half(agg2, l1[:NVP], xv, Wu[:1], Bu[:1])
    agg = _colsum_call(l2)
    q = _q_call(agg, W_q, jnp.reshape(b_q, (1, 1)), l2)
    return q[:NV]


# speedup vs baseline: 4.7775x; 1.1549x over previous
import functools

import jax
import jax.numpy as jnp
from jax import lax
from jax.experimental import pallas as pl
from jax.experimental.pallas import tpu as pltpu
from jax.experimental.pallas import tpu_sc as plsc

NV = 25000
NC = 25000
DEG = 16
OUT = 64
INIT_IN = 128
NW = 32
RB = 784
NVP = NW * RB
NT = 2 * NVP
CB = 56
LANES = 16


def _init_body(x_ref, w_ref, b_ref, last_ref):
    last_ref[...] = jnp.dot(x_ref[...], w_ref[0],
                            preferred_element_type=jnp.float32) + b_ref[0]


_init_call = pl.pallas_call(
    _init_body,
    grid=(NT // RB,),
    in_specs=[
        pl.BlockSpec((RB, INIT_IN), lambda i: (i, 0)),
        pl.BlockSpec((1, INIT_IN, OUT), lambda i: (i // (NVP // RB), 0, 0)),
        pl.BlockSpec((1, 1, OUT), lambda i: (i // (NVP // RB), 0, 0)),
    ],
    out_specs=pl.BlockSpec((RB, OUT), lambda i: (i, 0)),
    out_shape=jax.ShapeDtypeStruct((NT, OUT), jnp.float32),
)


def _update_body(agg_ref, last_ref, raw_ref, w_ref, b_ref, o_ref):
    cat = jnp.concatenate([agg_ref[...], last_ref[...], raw_ref[...]], axis=1)
    o_ref[...] = jnp.dot(cat, w_ref[0], preferred_element_type=jnp.float32) + b_ref[0]


def _make_update(n_rows):
    return pl.pallas_call(
        _update_body,
        grid=(n_rows // RB,),
        in_specs=[
            pl.BlockSpec((RB, OUT), lambda i: (i, 0)),
            pl.BlockSpec((RB, OUT), lambda i: (i, 0)),
            pl.BlockSpec((RB, INIT_IN), lambda i: (i, 0)),
            pl.BlockSpec((1, 2 * OUT + INIT_IN, OUT),
                         lambda i: (i // (NVP // RB), 0, 0)),
            pl.BlockSpec((1, 1, OUT), lambda i: (i // (NVP // RB), 0, 0)),
        ],
        out_specs=pl.BlockSpec((RB, OUT), lambda i: (i, 0)),
        out_shape=jax.ShapeDtypeStruct((n_rows, OUT), jnp.float32),
    )


_update_full = _make_update(NT)
_update_half = _make_update(NVP)


def _colsum_body(v_ref, o_ref):
    i = pl.program_id(0)

    @pl.when(i == 0)
    def _():
        o_ref[...] = jnp.zeros_like(o_ref)

    rows = lax.broadcasted_iota(jnp.int32, (RB, OUT), 0) + i * RB
    x = jnp.where(rows < NV, v_ref[...], 0.0)
    o_ref[0:1, 0:OUT] = o_ref[0:1, 0:OUT] + jnp.sum(x, axis=0, keepdims=True)


_colsum_call = pl.pallas_call(
    _colsum_body,
    grid=(NVP // RB,),
    in_specs=[pl.BlockSpec((RB, OUT), lambda i: (i, 0))],
    out_specs=pl.BlockSpec((8, 128), lambda i: (0, 0)),
    out_shape=jax.ShapeDtypeStruct((8, 128), jnp.float32),
)


def _q_body(agg_ref, wq_ref, bq_ref, v_ref, q_ref):
    agg_rep = jnp.broadcast_to(agg_ref[0:1, 0:OUT], (RB, OUT))
    cat = jnp.concatenate([agg_rep, v_ref[...]], axis=1)
    q_ref[...] = jnp.dot(cat, wq_ref[...],
                         preferred_element_type=jnp.float32) + bq_ref[0, 0]


_q_call = pl.pallas_call(
    _q_body,
    grid=(NVP // RB,),
    in_specs=[
        pl.BlockSpec((8, 128), lambda i: (0, 0)),
        pl.BlockSpec((2 * OUT, 1), lambda i: (0, 0)),
        pl.BlockSpec((1, 1), lambda i: (0, 0)),
        pl.BlockSpec((RB, OUT), lambda i: (i, 0)),
    ],
    out_specs=pl.BlockSpec((RB, 1), lambda i: (i, 0)),
    out_shape=jax.ShapeDtypeStruct((NVP, 1), jnp.float32),
)


def _make_gathersum(n_out):
    bw = n_out // NW
    nchunks = bw // CB
    mesh = plsc.VectorSubcoreMesh(core_axis_name="c", subcore_axis_name="s")

    @functools.partial(
        pl.kernel,
        out_type=jax.ShapeDtypeStruct((n_out, OUT), jnp.float32),
        mesh=mesh,
        scratch_types=[
            pltpu.VMEM((2, DEG, CB), jnp.int32),
            pltpu.VMEM((2, CB, OUT), jnp.float32),
            pltpu.VMEM((2, DEG, CB, OUT), jnp.float32),
            pltpu.SemaphoreType.DMA,
            pltpu.SemaphoreType.DMA,
            pltpu.SemaphoreType.DMA,
            pltpu.SemaphoreType.DMA,
            pltpu.SemaphoreType.DMA,
            pltpu.SemaphoreType.DMA,
        ],
        compiler_params=pltpu.CompilerParams(use_tc_tiling_on_sc=False),
    )
    def gathersum(table, idxt, out, idx_v, acc_v, gat_v,
                  isem0, isem1, gsem0, gsem1, osem0, osem1):
        isem = (isem0, isem1)
        gsem = (gsem0, gsem1)
        osem = (osem0, osem1)
        wid = lax.axis_index("s") * 2 + lax.axis_index("c")
        base = wid * bw

        def fire_idx(t):
            s = t % 2
            return pltpu.async_copy(
                idxt.at[:, pl.ds(base + t * CB, CB)], idx_v.at[s], isem[s])

        def fire_gat(t):
            s = t % 2
            return [pltpu.async_copy(table.at[idx_v.at[s, j]],
                                     gat_v.at[s, j], gsem[s])
                    for j in range(DEG)]

        def fire_out(t):
            s = t % 2
            return pltpu.async_copy(
                acc_v.at[s], out.at[pl.ds(base + t * CB, CB)], osem[s])

        def accum(t):
            s = t % 2

            def row_body(r, rc):
                for c in range(OUT // LANES):
                    sl = pl.ds(c * LANES, LANES)
                    v = gat_v[s, 0, r, sl]
                    for j in range(1, DEG):
                        v = v + gat_v[s, j, r, sl]
                    acc_v[s, r, sl] = v
                return rc

            lax.fori_loop(0, CB, row_body, 0)

        hi = {}
        hg = {}
        ho = {}
        hi[0] = fire_idx(0)
        hi[0].wait()
        hg[0] = fire_gat(0)
        if nchunks > 1:
            hi[1] = fire_idx(1)
        for t in range(nchunks):
            if t + 1 < nchunks:
                hi[t + 1].wait()
                hg[t + 1] = fire_gat(t + 1)
            for h in hg.pop(t):
                h.wait()
            if t + 2 < nchunks:
                hi[t + 2] = fire_idx(t + 2)
            if t - 2 >= 0:
                ho.pop(t - 2).wait()
            accum(t)
            ho[t] = fire_out(t)
        for t in list(ho):
            ho.pop(t).wait()

    return gathersum


_gathersum_full = _make_gathersum(NT)
_gathersum_half = _make_gathersum(NVP)


def kernel(x, var_constr_index, constr_var_index, W_iv, b_iv, W_ic, b_ic,
           W_v, b_v, W_c, b_c, W_q, b_q):
    pad = NVP - NV
    xv = jnp.pad(x[:NV], ((0, pad), (0, 0)))
    xc = jnp.pad(x[NV:], ((0, pad), (0, 0)))
    xp = jnp.concatenate([xv, xc], axis=0)

    Wi = jnp.stack([W_iv, W_ic])
    Bi = jnp.stack([b_iv[None, :], b_ic[None, :]])
    Wu = jnp.stack([W_v, W_c])
    Bu = jnp.stack([b_v[None, :], b_c[None, :]])

    idxt = jnp.concatenate([
        jnp.pad(var_constr_index + NVP, ((0, pad), (0, 0))),
        jnp.pad(constr_var_index, ((0, pad), (0, 0))),
    ], axis=0).T.astype(jnp.int32)

    last0 = _init_call(xp, Wi, Bi)
    agg1 = _gathersum_full(last0, idxt)
    l1 = _update_full(agg1, last0, xp, Wu, Bu)
    agg2 = _gathersum_half(l1, idxt)
    l2 = _update_half(agg2, l1[:NVP], xv, Wu[:1], Bu[:1])
    agg = _colsum_call(l2)
    q = _q_call(agg, W_q, jnp.reshape(b_q, (1, 1)), l2)
    return q[:NV]


# speedup vs baseline: 4.9000x; 1.0256x over previous
import functools

import jax
import jax.numpy as jnp
from jax import lax
from jax.experimental import pallas as pl
from jax.experimental.pallas import tpu as pltpu
from jax.experimental.pallas import tpu_sc as plsc

NV = 25000
NC = 25000
DEG = 16
OUT = 64
INIT_IN = 128
NW = 32
RB = 784
NVP = NW * RB
NT = 2 * NVP
CB = 56
LANES = 16


def _init_body(x_ref, w_ref, b_ref, last_ref):
    last_ref[...] = jnp.dot(x_ref[...], w_ref[0],
                            preferred_element_type=jnp.float32) + b_ref[0]


_init_call = pl.pallas_call(
    _init_body,
    grid=(NT // RB,),
    in_specs=[
        pl.BlockSpec((RB, INIT_IN), lambda i: (i, 0)),
        pl.BlockSpec((1, INIT_IN, OUT), lambda i: (i // (NVP // RB), 0, 0)),
        pl.BlockSpec((1, 1, OUT), lambda i: (i // (NVP // RB), 0, 0)),
    ],
    out_specs=pl.BlockSpec((RB, OUT), lambda i: (i, 0)),
    out_shape=jax.ShapeDtypeStruct((NT, OUT), jnp.float32),
)


def _update_body(agg_ref, last_ref, raw_ref, w_ref, b_ref, o_ref):
    cat = jnp.concatenate([agg_ref[...], last_ref[...], raw_ref[...]], axis=1)
    o_ref[...] = jnp.dot(cat, w_ref[0], preferred_element_type=jnp.float32) + b_ref[0]


def _make_update(n_rows):
    return pl.pallas_call(
        _update_body,
        grid=(n_rows // RB,),
        in_specs=[
            pl.BlockSpec((RB, OUT), lambda i: (i, 0)),
            pl.BlockSpec((RB, OUT), lambda i: (i, 0)),
            pl.BlockSpec((RB, INIT_IN), lambda i: (i, 0)),
            pl.BlockSpec((1, 2 * OUT + INIT_IN, OUT),
                         lambda i: (i // (NVP // RB), 0, 0)),
            pl.BlockSpec((1, 1, OUT), lambda i: (i // (NVP // RB), 0, 0)),
        ],
        out_specs=pl.BlockSpec((RB, OUT), lambda i: (i, 0)),
        out_shape=jax.ShapeDtypeStruct((n_rows, OUT), jnp.float32),
    )


_update_full = _make_update(NT)
_update_half = _make_update(NVP)


_NB = NVP // RB


def _q_body(wq_ref, bq_ref, v_ref, q_ref, agg_ref):
    i = pl.program_id(0)

    @pl.when(i == 0)
    def _():
        agg_ref[...] = jnp.zeros_like(agg_ref)

    @pl.when(i < _NB)
    def _():
        rows = lax.broadcasted_iota(jnp.int32, (RB, OUT), 0) + i * RB
        x = jnp.where(rows < NV, v_ref[...], 0.0)
        agg_ref[0:1, 0:OUT] = agg_ref[0:1, 0:OUT] + jnp.sum(x, axis=0,
                                                            keepdims=True)

    @pl.when(i >= _NB)
    def _():
        agg_rep = jnp.broadcast_to(agg_ref[0:1, 0:OUT], (RB, OUT))
        cat = jnp.concatenate([agg_rep, v_ref[...]], axis=1)
        q_ref[...] = jnp.dot(cat, wq_ref[...],
                             preferred_element_type=jnp.float32) + bq_ref[0, 0]


_q_call = pl.pallas_call(
    _q_body,
    grid=(2 * _NB,),
    in_specs=[
        pl.BlockSpec((2 * OUT, 1), lambda i: (0, 0)),
        pl.BlockSpec((1, 1), lambda i: (0, 0)),
        pl.BlockSpec((RB, OUT), lambda i: (jnp.where(i < _NB, i, i - _NB), 0)),
    ],
    out_specs=pl.BlockSpec((RB, 1), lambda i: (jnp.where(i < _NB, 0, i - _NB), 0)),
    out_shape=jax.ShapeDtypeStruct((NVP, 1), jnp.float32),
    scratch_shapes=[pltpu.VMEM((8, 128), jnp.float32)],
)


def _make_gathersum(n_out):
    bw = n_out // NW
    nchunks = bw // CB
    mesh = plsc.VectorSubcoreMesh(core_axis_name="c", subcore_axis_name="s")

    @functools.partial(
        pl.kernel,
        out_type=jax.ShapeDtypeStruct((n_out, OUT), jnp.float32),
        mesh=mesh,
        scratch_types=[
            pltpu.VMEM((2, DEG, CB), jnp.int32),
            pltpu.VMEM((2, CB, OUT), jnp.float32),
            pltpu.VMEM((2, DEG, CB, OUT), jnp.float32),
            pltpu.SemaphoreType.DMA,
            pltpu.SemaphoreType.DMA,
            pltpu.SemaphoreType.DMA,
            pltpu.SemaphoreType.DMA,
            pltpu.SemaphoreType.DMA,
            pltpu.SemaphoreType.DMA,
        ],
        compiler_params=pltpu.CompilerParams(use_tc_tiling_on_sc=False),
    )
    def gathersum(table, idxt, out, idx_v, acc_v, gat_v,
                  isem0, isem1, gsem0, gsem1, osem0, osem1):
        isem = (isem0, isem1)
        gsem = (gsem0, gsem1)
        osem = (osem0, osem1)
        wid = lax.axis_index("s") * 2 + lax.axis_index("c")
        base = wid * bw

        def fire_idx(t):
            s = t % 2
            return pltpu.async_copy(
                idxt.at[:, pl.ds(base + t * CB, CB)], idx_v.at[s], isem[s])

        def fire_gat(t):
            s = t % 2
            return [pltpu.async_copy(table.at[idx_v.at[s, j]],
                                     gat_v.at[s, j], gsem[s])
                    for j in range(DEG)]

        def fire_out(t):
            s = t % 2
            return pltpu.async_copy(
                acc_v.at[s], out.at[pl.ds(base + t * CB, CB)], osem[s])

        def accum(t):
            s = t % 2

            def row_body(r, rc):
                for c in range(OUT // LANES):
                    sl = pl.ds(c * LANES, LANES)
                    v = gat_v[s, 0, r, sl]
                    for j in range(1, DEG):
                        v = v + gat_v[s, j, r, sl]
                    acc_v[s, r, sl] = v
                return rc

            lax.fori_loop(0, CB, row_body, 0)

        hi = {}
        hg = {}
        ho = {}
        hi[0] = fire_idx(0)
        hi[0].wait()
        hg[0] = fire_gat(0)
        if nchunks > 1:
            hi[1] = fire_idx(1)
        for t in range(nchunks):
            if t + 1 < nchunks:
                hi[t + 1].wait()
                hg[t + 1] = fire_gat(t + 1)
            for h in hg.pop(t):
                h.wait()
            if t + 2 < nchunks:
                hi[t + 2] = fire_idx(t + 2)
            if t - 2 >= 0:
                ho.pop(t - 2).wait()
            accum(t)
            ho[t] = fire_out(t)
        for t in list(ho):
            ho.pop(t).wait()

    return gathersum


_gathersum_full = _make_gathersum(NT)
_gathersum_half = _make_gathersum(NVP)


def kernel(x, var_constr_index, constr_var_index, W_iv, b_iv, W_ic, b_ic,
           W_v, b_v, W_c, b_c, W_q, b_q):
    pad = NVP - NV
    xv = jnp.pad(x[:NV], ((0, pad), (0, 0)))
    xc = jnp.pad(x[NV:], ((0, pad), (0, 0)))
    xp = jnp.concatenate([xv, xc], axis=0)

    Wi = jnp.stack([W_iv, W_ic])
    Bi = jnp.stack([b_iv[None, :], b_ic[None, :]])
    Wu = jnp.stack([W_v, W_c])
    Bu = jnp.stack([b_v[None, :], b_c[None, :]])

    idxt = jnp.concatenate([
        jnp.pad(var_constr_index + NVP, ((0, pad), (0, 0))),
        jnp.pad(constr_var_index, ((0, pad), (0, 0))),
    ], axis=0).T.astype(jnp.int32)

    last0 = _init_call(xp, Wi, Bi)
    agg1 = _gathersum_full(last0, idxt)
    l1 = _update_full(agg1, last0, xp, Wu, Bu)
    agg2 = _gathersum_half(l1, idxt)
    l2 = _update_half(agg2, l1, xv, Wu, Bu)
    q = _q_call(W_q, jnp.reshape(b_q, (1, 1)), l2)
    return q[:NV]


# speedup vs baseline: 4.9987x; 1.0201x over previous
import functools

import jax
import jax.numpy as jnp
from jax import lax
from jax.experimental import pallas as pl
from jax.experimental.pallas import tpu as pltpu
from jax.experimental.pallas import tpu_sc as plsc

NV = 25000
NC = 25000
DEG = 16
OUT = 64
INIT_IN = 128
NW = 32
RB = 784
NVP = NW * RB
NT = 2 * NVP
CB = 56
LANES = 16


def _init_body(x_ref, w_ref, b_ref, last_ref):
    last_ref[...] = jnp.dot(x_ref[...], w_ref[0],
                            preferred_element_type=jnp.float32) + b_ref[0]


_init_call = pl.pallas_call(
    _init_body,
    grid=(NT // RB,),
    in_specs=[
        pl.BlockSpec((RB, INIT_IN), lambda i: (i, 0)),
        pl.BlockSpec((1, INIT_IN, OUT), lambda i: (i // (NVP // RB), 0, 0)),
        pl.BlockSpec((1, 1, OUT), lambda i: (i // (NVP // RB), 0, 0)),
    ],
    out_specs=pl.BlockSpec((RB, OUT), lambda i: (i, 0)),
    out_shape=jax.ShapeDtypeStruct((NT, OUT), jnp.float32),
)


def _update_body(agg_ref, last_ref, raw_ref, w_ref, b_ref, o_ref):
    cat = jnp.concatenate([agg_ref[...], last_ref[...], raw_ref[...]], axis=1)
    o_ref[...] = jnp.dot(cat, w_ref[0], preferred_element_type=jnp.float32) + b_ref[0]


def _make_update(n_rows):
    return pl.pallas_call(
        _update_body,
        grid=(n_rows // RB,),
        in_specs=[
            pl.BlockSpec((RB, OUT), lambda i: (i, 0)),
            pl.BlockSpec((RB, OUT), lambda i: (i, 0)),
            pl.BlockSpec((RB, INIT_IN), lambda i: (i, 0)),
            pl.BlockSpec((1, 2 * OUT + INIT_IN, OUT),
                         lambda i: (i // (NVP // RB), 0, 0)),
            pl.BlockSpec((1, 1, OUT), lambda i: (i // (NVP // RB), 0, 0)),
        ],
        out_specs=pl.BlockSpec((RB, OUT), lambda i: (i, 0)),
        out_shape=jax.ShapeDtypeStruct((n_rows, OUT), jnp.float32),
    )


_update_half = _make_update(NVP)


def _make_update_off(block_off, widx):
    return pl.pallas_call(
        _update_body,
        grid=(NVP // RB,),
        in_specs=[
            pl.BlockSpec((RB, OUT), lambda i: (block_off + i, 0)),
            pl.BlockSpec((RB, OUT), lambda i: (block_off + i, 0)),
            pl.BlockSpec((RB, INIT_IN), lambda i: (block_off + i, 0)),
            pl.BlockSpec((1, 2 * OUT + INIT_IN, OUT), lambda i: (widx, 0, 0)),
            pl.BlockSpec((1, 1, OUT), lambda i: (widx, 0, 0)),
        ],
        out_specs=pl.BlockSpec((RB, OUT), lambda i: (i, 0)),
        out_shape=jax.ShapeDtypeStruct((NVP, OUT), jnp.float32),
    )


_upd_con = _make_update_off(NVP // RB, 1)
_upd_var = _make_update_off(0, 0)


_NB = NVP // RB


def _q_body(wq_ref, bq_ref, v_ref, q_ref, agg_ref):
    i = pl.program_id(0)

    @pl.when(i == 0)
    def _():
        agg_ref[...] = jnp.zeros_like(agg_ref)

    @pl.when(i < _NB)
    def _():
        rows = lax.broadcasted_iota(jnp.int32, (RB, OUT), 0) + i * RB
        x = jnp.where(rows < NV, v_ref[...], 0.0)
        agg_ref[0:1, 0:OUT] = agg_ref[0:1, 0:OUT] + jnp.sum(x, axis=0,
                                                            keepdims=True)

    @pl.when(i >= _NB)
    def _():
        agg_rep = jnp.broadcast_to(agg_ref[0:1, 0:OUT], (RB, OUT))
        cat = jnp.concatenate([agg_rep, v_ref[...]], axis=1)
        q_ref[...] = jnp.dot(cat, wq_ref[...],
                             preferred_element_type=jnp.float32) + bq_ref[0, 0]


_q_call = pl.pallas_call(
    _q_body,
    grid=(2 * _NB,),
    in_specs=[
        pl.BlockSpec((2 * OUT, 1), lambda i: (0, 0)),
        pl.BlockSpec((1, 1), lambda i: (0, 0)),
        pl.BlockSpec((RB, OUT), lambda i: (jnp.where(i < _NB, i, i - _NB), 0)),
    ],
    out_specs=pl.BlockSpec((RB, 1), lambda i: (jnp.where(i < _NB, 0, i - _NB), 0)),
    out_shape=jax.ShapeDtypeStruct((NVP, 1), jnp.float32),
    scratch_shapes=[pltpu.VMEM((8, 128), jnp.float32)],
)


def _make_gathersum(n_out):
    bw = n_out // NW
    nchunks = bw // CB
    mesh = plsc.VectorSubcoreMesh(core_axis_name="c", subcore_axis_name="s")

    @functools.partial(
        pl.kernel,
        out_type=jax.ShapeDtypeStruct((n_out, OUT), jnp.float32),
        mesh=mesh,
        scratch_types=[
            pltpu.VMEM((2, DEG, CB), jnp.int32),
            pltpu.VMEM((2, CB, OUT), jnp.float32),
            pltpu.VMEM((2, DEG, CB, OUT), jnp.float32),
            pltpu.SemaphoreType.DMA,
            pltpu.SemaphoreType.DMA,
            pltpu.SemaphoreType.DMA,
            pltpu.SemaphoreType.DMA,
            pltpu.SemaphoreType.DMA,
            pltpu.SemaphoreType.DMA,
        ],
        compiler_params=pltpu.CompilerParams(use_tc_tiling_on_sc=False),
    )
    def gathersum(table, idxt, out, idx_v, acc_v, gat_v,
                  isem0, isem1, gsem0, gsem1, osem0, osem1):
        isem = (isem0, isem1)
        gsem = (gsem0, gsem1)
        osem = (osem0, osem1)
        wid = lax.axis_index("s") * 2 + lax.axis_index("c")
        base = wid * bw

        def fire_idx(t):
            s = t % 2
            return pltpu.async_copy(
                idxt.at[:, pl.ds(base + t * CB, CB)], idx_v.at[s], isem[s])

        def fire_gat(t):
            s = t % 2
            return [pltpu.async_copy(table.at[idx_v.at[s, j]],
                                     gat_v.at[s, j], gsem[s])
                    for j in range(DEG)]

        def fire_out(t):
            s = t % 2
            return pltpu.async_copy(
                acc_v.at[s], out.at[pl.ds(base + t * CB, CB)], osem[s])

        def accum(t):
            s = t % 2

            def row_body(r, rc):
                for c in range(OUT // LANES):
                    sl = pl.ds(c * LANES, LANES)
                    v = gat_v[s, 0, r, sl]
                    for j in range(1, DEG):
                        v = v + gat_v[s, j, r, sl]
                    acc_v[s, r, sl] = v
                return rc

            lax.fori_loop(0, CB, row_body, 0)

        hi = {}
        hg = {}
        ho = {}
        hi[0] = fire_idx(0)
        hi[0].wait()
        hg[0] = fire_gat(0)
        if nchunks > 1:
            hi[1] = fire_idx(1)
        for t in range(nchunks):
            if t + 1 < nchunks:
                hi[t + 1].wait()
                hg[t + 1] = fire_gat(t + 1)
            for h in hg.pop(t):
                h.wait()
            if t + 2 < nchunks:
                hi[t + 2] = fire_idx(t + 2)
            if t - 2 >= 0:
                ho.pop(t - 2).wait()
            accum(t)
            ho[t] = fire_out(t)
        for t in list(ho):
            ho.pop(t).wait()

    return gathersum


_gathersum_full = _make_gathersum(NT)
_gathersum_half = _make_gathersum(NVP)


def kernel(x, var_constr_index, constr_var_index, W_iv, b_iv, W_ic, b_ic,
           W_v, b_v, W_c, b_c, W_q, b_q):
    pad = NVP - NV
    xv = jnp.pad(x[:NV], ((0, pad), (0, 0)))
    xc = jnp.pad(x[NV:], ((0, pad), (0, 0)))
    xp = jnp.concatenate([xv, xc], axis=0)

    Wi = jnp.stack([W_iv, W_ic])
    Bi = jnp.stack([b_iv[None, :], b_ic[None, :]])
    Wu = jnp.stack([W_v, W_c])
    Bu = jnp.stack([b_v[None, :], b_c[None, :]])

    idxt = jnp.concatenate([
        jnp.pad(var_constr_index + NVP, ((0, pad), (0, 0))),
        jnp.pad(constr_var_index, ((0, pad), (0, 0))),
    ], axis=0).T.astype(jnp.int32)
    idxt2 = jnp.pad(var_constr_index,
                    ((0, pad), (0, 0))).T.astype(jnp.int32)

    last0 = _init_call(xp, Wi, Bi)
    agg1 = _gathersum_full(last0, idxt)
    lc1 = _upd_con(agg1, last0, xp, Wu, Bu)
    agg2 = _gathersum_half(lc1, idxt2)
    lv1 = _upd_var(agg1, last0, xp, Wu, Bu)
    l2 = _update_half(agg2, lv1, xv, Wu, Bu)
    q = _q_call(W_q, jnp.reshape(b_q, (1, 1)), l2)
    return q[:NV]
